# Initial kernel scaffold; baseline (speedup 1.0000x reference)
import jax
import jax.numpy as jnp
from jax.experimental import pallas as pl


def kernel(pos, encoding):
    raise NotImplementedError("write your pallas kernel here")


# speedup vs baseline: 2.1760x; 2.1760x over previous
import functools

import jax
import jax.numpy as jnp
from jax import lax
from jax.experimental import pallas as pl
from jax.experimental.pallas import tpu as pltpu
from jax.experimental.pallas import tpu_sc as plsc

_NC = 2
_NS = 16
_NW = _NC * _NS

_B = 32768
_D = 1024
_BPW = _B // _NW
_C = 64


def _gather_body(pos_hbm, enc_hbm, out_hbm, idx_v, rows_v, sem):
    c = lax.axis_index("c")
    s = lax.axis_index("s")
    wid = s * _NC + c
    base = pl.multiple_of(wid * _BPW, _BPW)
    pltpu.sync_copy(pos_hbm.at[pl.ds(base, _BPW)], idx_v)

    def step(g, carry):
        off = pl.multiple_of(g * _C, _C)
        pltpu.async_copy(enc_hbm.at[idx_v.at[pl.ds(off, _C)]], rows_v, sem).wait()
        pltpu.sync_copy(rows_v, out_hbm.at[pl.ds(base + off, _C)])
        return carry

    lax.fori_loop(0, _BPW // _C, step, 0)


@functools.partial(jax.jit, static_argnames=())
def _gather(pos_flat, encoding):
    mesh = plsc.VectorSubcoreMesh(core_axis_name="c", subcore_axis_name="s")
    run = pl.kernel(
        _gather_body,
        out_type=jax.ShapeDtypeStruct((_B, _D), jnp.float32),
        mesh=mesh,
        scratch_types=[
            pltpu.VMEM((_BPW,), jnp.int32),
            pltpu.VMEM((_C, _D), jnp.float32),
            pltpu.SemaphoreType.DMA,
        ],
    )
    return run(pos_flat, encoding)


def kernel(pos, encoding):
    b, s = pos.shape
    out = _gather(pos.reshape(-1), encoding)
    return out.reshape(b, s, encoding.shape[1])


# speedup vs baseline: 2.3252x; 1.0686x over previous
import functools

import jax
import jax.numpy as jnp
from jax import lax
from jax.experimental import pallas as pl
from jax.experimental.pallas import tpu as pltpu
from jax.experimental.pallas import tpu_sc as plsc

_NC = 2
_NS = 16
_NW = _NC * _NS

_B = 32768
_D = 1024
_BPW = _B // _NW
_C = 16
_NBUF = 4
_G = _BPW // _C
_T = _G // _NBUF


def _gather_body(pos_hbm, enc_hbm, out_hbm, idx_v, *scratch):
    rows = scratch[:_NBUF]
    gsems = scratch[_NBUF:2 * _NBUF]
    ssems = scratch[2 * _NBUF:3 * _NBUF]

    c = lax.axis_index("c")
    s = lax.axis_index("s")
    wid = s * _NC + c
    base = pl.multiple_of(wid * _BPW, _BPW)

    pltpu.sync_copy(pos_hbm.at[pl.ds(base, _BPW)], idx_v)

    def start_gather(g_off, b):
        pltpu.async_copy(enc_hbm.at[idx_v.at[pl.ds(g_off, _C)]], rows[b], gsems[b])

    def wait_gather(b):
        pltpu.make_async_copy(enc_hbm.at[idx_v.at[pl.ds(0, _C)]], rows[b],
                              gsems[b]).wait()

    def start_store(g_off, b):
        pltpu.async_copy(rows[b], out_hbm.at[pl.ds(base + g_off, _C)], ssems[b])

    def drain_store(b):
        pltpu.make_async_copy(rows[b], out_hbm.at[pl.ds(0, _C)], ssems[b]).wait()

    for b in range(_NBUF):
        start_gather(b * _C, b)
    for b in range(_NBUF):
        wait_gather(b)
        start_store(b * _C, b)

    def ring_round(t, carry):
        for b in range(_NBUF):
            off = pl.multiple_of((t * _NBUF + b) * _C, _C)
            drain_store(b)
            start_gather(off, b)
        for b in range(_NBUF):
            off = pl.multiple_of((t * _NBUF + b) * _C, _C)
            wait_gather(b)
            start_store(off, b)
        return carry

    lax.fori_loop(1, _T, ring_round, 0)

    for b in range(_NBUF):
        drain_store(b)


@functools.partial(jax.jit, static_argnames=())
def _gather(pos_flat, encoding):
    mesh = plsc.VectorSubcoreMesh(core_axis_name="c", subcore_axis_name="s")
    run = pl.kernel(
        _gather_body,
        out_type=jax.ShapeDtypeStruct((_B, _D), jnp.float32),
        mesh=mesh,
        scratch_types=(
            [pltpu.VMEM((_BPW,), jnp.int32)]
            + [pltpu.VMEM((_C, _D), jnp.float32) for _ in range(_NBUF)]
            + [pltpu.SemaphoreType.DMA for _ in range(2 * _NBUF)]
        ),
    )
    return run(pos_flat, encoding)


def kernel(pos, encoding):
    b, s = pos.shape
    out = _gather(pos.reshape(-1), encoding)
    return out.reshape(b, s, encoding.shape[1])
